# baseline (speedup 1.0000x reference)
import functools

import jax
import jax.numpy as jnp
from jax import lax
from jax.experimental import pallas as pl
from jax.experimental.pallas import tpu as pltpu
from jax.experimental.pallas import tpu_sc as plsc

_D = 64
_NC = 2
_NS = 16
_NW = _NC * _NS
_CHUNK = 800


def _embed_body(x_hbm, table_hbm, out_hbm, idx_v, rows_v, gsem):
    n_total = x_hbm.shape[0]
    b_per_w = n_total // _NW
    n_chunks = b_per_w // _CHUNK
    wid = lax.axis_index("s") * _NC + lax.axis_index("c")
    base = wid * b_per_w

    @pl.loop(0, n_chunks)
    def _chunk(i):
        off = base + i * _CHUNK
        pltpu.sync_copy(x_hbm.at[pl.ds(off, _CHUNK)], idx_v)
        pltpu.async_copy(table_hbm.at[idx_v], rows_v, gsem).wait()
        pltpu.sync_copy(rows_v, out_hbm.at[pl.ds(off, _CHUNK)])


@jax.jit
def _embed(flat_x, embed_weight):
    n = flat_x.shape[0]
    k = pl.kernel(
        _embed_body,
        out_type=jax.ShapeDtypeStruct((n, _D), jnp.float32),
        mesh=plsc.VectorSubcoreMesh(core_axis_name="c", subcore_axis_name="s"),
        scratch_types=[
            pltpu.VMEM((_CHUNK,), jnp.int32),
            pltpu.VMEM((_CHUNK, _D), jnp.float32),
            pltpu.SemaphoreType.DMA,
        ],
        compiler_params=pltpu.CompilerParams(use_tc_tiling_on_sc=False),
    )
    return k(flat_x, embed_weight)


def kernel(x, embed_weight):
    b, h = x.shape
    out = _embed(x.reshape(b * h), embed_weight)
    return out.reshape(b, h, _D)


# speedup vs baseline: 1.0236x; 1.0236x over previous
import jax
import jax.numpy as jnp
from jax import lax
from jax.experimental import pallas as pl
from jax.experimental.pallas import tpu as pltpu
from jax.experimental.pallas import tpu_sc as plsc

_D = 64
_NC = 2
_NS = 16
_NW = _NC * _NS
_CHUNK = 800
_NBUF = 2


def _embed_body(x_hbm, table_hbm, out_hbm, idx_v, rows_v, isems, gsems, osems):
    n_total = x_hbm.shape[0]
    b_per_w = n_total // _NW
    n_chunks = b_per_w // _CHUNK
    wid = lax.axis_index("s") * _NC + lax.axis_index("c")
    base = wid * b_per_w

    for b in range(_NBUF):
        pltpu.async_copy(
            x_hbm.at[pl.ds(base + b * _CHUNK, _CHUNK)], idx_v.at[b], isems.at[b]
        )

    @pl.loop(0, n_chunks // _NBUF)
    def _outer(t):
        g0 = t * _NBUF
        for b in range(_NBUF):
            g = g0 + b
            off = base + g * _CHUNK

            @pl.when(t > 0)
            def _():
                pltpu.make_async_copy(
                    rows_v.at[b], out_hbm.at[pl.ds(base, _CHUNK)], osems.at[b]
                ).wait()

            pltpu.make_async_copy(
                x_hbm.at[pl.ds(off, _CHUNK)], idx_v.at[b], isems.at[b]
            ).wait()
            pltpu.async_copy(
                table_hbm.at[idx_v.at[b]], rows_v.at[b], gsems.at[b]
            ).wait()

            pltpu.async_copy(rows_v.at[b], out_hbm.at[pl.ds(off, _CHUNK)], osems.at[b])

            @pl.when(g + _NBUF < n_chunks)
            def _():
                off2 = base + (g + _NBUF) * _CHUNK
                pltpu.async_copy(
                    x_hbm.at[pl.ds(off2, _CHUNK)], idx_v.at[b], isems.at[b]
                )

    for b in range(_NBUF):
        pltpu.make_async_copy(
            rows_v.at[b], out_hbm.at[pl.ds(base, _CHUNK)], osems.at[b]
        ).wait()


@jax.jit
def _embed(flat_x, embed_weight):
    n = flat_x.shape[0]
    k = pl.kernel(
        _embed_body,
        out_type=jax.ShapeDtypeStruct((n, _D), jnp.float32),
        mesh=plsc.VectorSubcoreMesh(core_axis_name="c", subcore_axis_name="s"),
        scratch_types=[
            pltpu.VMEM((_NBUF, _CHUNK), jnp.int32),
            pltpu.VMEM((_NBUF, _CHUNK, _D), jnp.float32),
            pltpu.SemaphoreType.DMA((_NBUF,)),
            pltpu.SemaphoreType.DMA((_NBUF,)),
            pltpu.SemaphoreType.DMA((_NBUF,)),
        ],
        compiler_params=pltpu.CompilerParams(use_tc_tiling_on_sc=False),
    )
    return k(flat_x, embed_weight)


def kernel(x, embed_weight):
    b, h = x.shape
    out = _embed(x.reshape(b * h), embed_weight)
    return out.reshape(b, h, _D)


# speedup vs baseline: 1.2960x; 1.2661x over previous
import functools

import jax
import jax.numpy as jnp
from jax import lax
from jax.experimental import pallas as pl
from jax.experimental.pallas import tpu as pltpu
from jax.experimental.pallas import tpu_sc as plsc

_VOCAB = 1000000
_TCOLS = 2048
_NBLK = 244
_SPLIT = _NBLK * _TCOLS
_MID = 2 * _SPLIT
_PROWS = _SPLIT + (_VOCAB - _MID)
_D = 64
_NC = 2
_NS = 16
_NW = _NC * _NS
_CHUNK = 800
_NBUF = 2
def _repack_body(lo_ref, hi_ref, out_ref):
    out_ref[:, 0:_D] = jnp.swapaxes(lo_ref[...], 0, 1)
    out_ref[:, _D:128] = jnp.swapaxes(hi_ref[...], 0, 1)


@jax.jit
def _repack(wt):
    return pl.pallas_call(
        _repack_body,
        grid=(_NBLK + 1,),
        in_specs=[
            pl.BlockSpec((_D, _TCOLS), lambda g: (0, jnp.where(g < _NBLK, g, 2 * _NBLK))),
            pl.BlockSpec((_D, _TCOLS), lambda g: (0, g + _NBLK)),
        ],
        out_specs=pl.BlockSpec((_TCOLS, 128), lambda g: (g, 0)),
        out_shape=jax.ShapeDtypeStruct((_PROWS, 128), jnp.float32),
    )(wt, wt)


def _embed_body(x_hbm, table_hbm, out_hbm, idx_v, pidx_v, rows_v, isems, gsems, osems):
    n_total = x_hbm.shape[0]
    b_per_w = n_total // _NW
    n_chunks = b_per_w // _CHUNK
    wid = lax.axis_index("s") * _NC + lax.axis_index("c")
    base = wid * b_per_w

    for b in range(_NBUF):
        pltpu.async_copy(
            x_hbm.at[pl.ds(base + b * _CHUNK, _CHUNK)], idx_v.at[b], isems.at[b]
        )

    @pl.loop(0, n_chunks // _NBUF)
    def _outer(t):
        g0 = t * _NBUF
        for b in range(_NBUF):
            g = g0 + b
            off = base + g * _CHUNK

            @pl.when(t > 0)
            def _():
                pltpu.make_async_copy(
                    rows_v.at[b], out_hbm.at[pl.ds(base, _CHUNK)], osems.at[b]
                ).wait()

            pltpu.make_async_copy(
                x_hbm.at[pl.ds(off, _CHUNK)], idx_v.at[b], isems.at[b]
            ).wait()

            @pl.loop(0, _CHUNK // 16)
            def _remap(j):
                v = idx_v[b, pl.ds(j * 16, 16)]
                q = jnp.where(
                    v < _SPLIT,
                    v + v,
                    jnp.where(
                        v < _MID,
                        (v - _SPLIT) * 2 + 1,
                        _MID + (v - _MID) * 2,
                    ),
                )
                pidx_v[b, pl.ds(j * 16, 16)] = q

            pltpu.async_copy(
                table_hbm.at[pidx_v.at[b]], rows_v.at[b], gsems.at[b]
            ).wait()

            pltpu.async_copy(rows_v.at[b], out_hbm.at[pl.ds(off, _CHUNK)], osems.at[b])

            @pl.when(g + _NBUF < n_chunks)
            def _():
                off2 = base + (g + _NBUF) * _CHUNK
                pltpu.async_copy(
                    x_hbm.at[pl.ds(off2, _CHUNK)], idx_v.at[b], isems.at[b]
                )

    for b in range(_NBUF):
        pltpu.make_async_copy(
            rows_v.at[b], out_hbm.at[pl.ds(base, _CHUNK)], osems.at[b]
        ).wait()


@jax.jit
def _embed(flat_x, table_lin):
    n = flat_x.shape[0]
    k = pl.kernel(
        _embed_body,
        out_type=jax.ShapeDtypeStruct((n, _D), jnp.float32),
        mesh=plsc.VectorSubcoreMesh(core_axis_name="c", subcore_axis_name="s"),
        scratch_types=[
            pltpu.VMEM((_NBUF, _CHUNK), jnp.int32),
            pltpu.VMEM((_NBUF, _CHUNK), jnp.int32),
            pltpu.VMEM((_NBUF, _CHUNK, _D), jnp.float32),
            pltpu.SemaphoreType.DMA((_NBUF,)),
            pltpu.SemaphoreType.DMA((_NBUF,)),
            pltpu.SemaphoreType.DMA((_NBUF,)),
        ],
        compiler_params=pltpu.CompilerParams(use_tc_tiling_on_sc=False),
    )
    return k(flat_x, table_lin)


def kernel(x, embed_weight):
    b, h = x.shape
    table2 = _repack(embed_weight.T).reshape(2 * _PROWS, _D)
    out = _embed(x.reshape(b * h), table2)
    return out.reshape(b, h, _D)


# speedup vs baseline: 1.3909x; 1.0732x over previous
import functools

import jax
import jax.numpy as jnp
from jax import lax
from jax.experimental import pallas as pl
from jax.experimental.pallas import tpu as pltpu
from jax.experimental.pallas import tpu_sc as plsc

_VOCAB = 1000000
_TCOLS = 4096
_NBLK = 122
_SPLIT = _NBLK * _TCOLS
_MID = 2 * _SPLIT
_PROWS = _SPLIT + (_VOCAB - _MID)
_D = 64
_NC = 2
_NS = 16
_NW = _NC * _NS
_CHUNK = 800
_NBUF = 2
def _repack_body(lo_ref, hi_ref, out_ref):
    out_ref[:, 0:_D] = jnp.swapaxes(lo_ref[...], 0, 1)
    out_ref[:, _D:128] = jnp.swapaxes(hi_ref[...], 0, 1)


@jax.jit
def _repack(wt):
    return pl.pallas_call(
        _repack_body,
        grid=(_NBLK + 1,),
        in_specs=[
            pl.BlockSpec((_D, _TCOLS), lambda g: (0, jnp.where(g < _NBLK, g, 2 * _NBLK))),
            pl.BlockSpec((_D, _TCOLS), lambda g: (0, g + _NBLK)),
        ],
        out_specs=pl.BlockSpec((_TCOLS, 128), lambda g: (g, 0)),
        out_shape=jax.ShapeDtypeStruct((_PROWS, 128), jnp.float32),
    )(wt, wt)


def _embed_body(x_hbm, table_hbm, out_hbm, idx_v, pidx_v, rows_v, isems, gsems, osems):
    n_total = x_hbm.shape[0]
    b_per_w = n_total // _NW
    n_chunks = b_per_w // _CHUNK
    wid = lax.axis_index("s") * _NC + lax.axis_index("c")
    base = wid * b_per_w

    for b in range(_NBUF):
        pltpu.async_copy(
            x_hbm.at[pl.ds(base + b * _CHUNK, _CHUNK)], idx_v.at[b], isems.at[b]
        )

    @pl.loop(0, n_chunks // _NBUF)
    def _outer(t):
        g0 = t * _NBUF
        for b in range(_NBUF):
            g = g0 + b
            off = base + g * _CHUNK

            @pl.when(t > 0)
            def _():
                pltpu.make_async_copy(
                    rows_v.at[b], out_hbm.at[pl.ds(base, _CHUNK)], osems.at[b]
                ).wait()

            pltpu.make_async_copy(
                x_hbm.at[pl.ds(off, _CHUNK)], idx_v.at[b], isems.at[b]
            ).wait()

            @pl.loop(0, _CHUNK // 16)
            def _remap(j):
                v = idx_v[b, pl.ds(j * 16, 16)]
                q = jnp.where(
                    v < _SPLIT,
                    v + v,
                    jnp.where(
                        v < _MID,
                        (v - _SPLIT) * 2 + 1,
                        _MID + (v - _MID) * 2,
                    ),
                )
                pidx_v[b, pl.ds(j * 16, 16)] = q

            pltpu.async_copy(
                table_hbm.at[pidx_v.at[b]], rows_v.at[b], gsems.at[b]
            ).wait()

            pltpu.async_copy(rows_v.at[b], out_hbm.at[pl.ds(off, _CHUNK)], osems.at[b])

            @pl.when(g + _NBUF < n_chunks)
            def _():
                off2 = base + (g + _NBUF) * _CHUNK
                pltpu.async_copy(
                    x_hbm.at[pl.ds(off2, _CHUNK)], idx_v.at[b], isems.at[b]
                )

    for b in range(_NBUF):
        pltpu.make_async_copy(
            rows_v.at[b], out_hbm.at[pl.ds(base, _CHUNK)], osems.at[b]
        ).wait()


@jax.jit
def _embed(flat_x, table_lin):
    n = flat_x.shape[0]
    k = pl.kernel(
        _embed_body,
        out_type=jax.ShapeDtypeStruct((n, _D), jnp.float32),
        mesh=plsc.VectorSubcoreMesh(core_axis_name="c", subcore_axis_name="s"),
        scratch_types=[
            pltpu.VMEM((_NBUF, _CHUNK), jnp.int32),
            pltpu.VMEM((_NBUF, _CHUNK), jnp.int32),
            pltpu.VMEM((_NBUF, _CHUNK, _D), jnp.float32),
            pltpu.SemaphoreType.DMA((_NBUF,)),
            pltpu.SemaphoreType.DMA((_NBUF,)),
            pltpu.SemaphoreType.DMA((_NBUF,)),
        ],
        compiler_params=pltpu.CompilerParams(use_tc_tiling_on_sc=False),
    )
    return k(flat_x, table_lin)


def kernel(x, embed_weight):
    b, h = x.shape
    table2 = _repack(embed_weight.T).reshape(2 * _PROWS, _D)
    out = _embed(x.reshape(b * h), table2)
    return out.reshape(b, h, _D)


# speedup vs baseline: 1.4452x; 1.0390x over previous
import functools

import jax
import jax.numpy as jnp
from jax import lax
from jax.experimental import pallas as pl
from jax.experimental.pallas import tpu as pltpu
from jax.experimental.pallas import tpu_sc as plsc

_VOCAB = 1000000
_TCOLS = 8192
_NBLK = 61
_SPLIT = _NBLK * _TCOLS
_MID = 2 * _SPLIT
_PROWS = _SPLIT + (_VOCAB - _MID)
_D = 64
_NC = 2
_NS = 16
_NW = _NC * _NS
_CHUNK = 800
_NBUF = 2
def _repack_body(lo_ref, hi_ref, out_ref):
    out_ref[...] = jnp.concatenate(
        [jnp.swapaxes(lo_ref[...], 0, 1), jnp.swapaxes(hi_ref[...], 0, 1)], axis=1
    )


@jax.jit
def _repack(wt):
    return pl.pallas_call(
        _repack_body,
        grid=(_NBLK + 1,),
        in_specs=[
            pl.BlockSpec((_D, _TCOLS), lambda g: (0, jnp.where(g < _NBLK, g, 2 * _NBLK))),
            pl.BlockSpec((_D, _TCOLS), lambda g: (0, g + _NBLK)),
        ],
        out_specs=pl.BlockSpec((_TCOLS, 128), lambda g: (g, 0)),
        out_shape=jax.ShapeDtypeStruct((_PROWS, 128), jnp.float32),
    )(wt, wt)


def _embed_body(x_hbm, table_hbm, out_hbm, idx_v, pidx_v, rows_v, isems, gsems, osems):
    n_total = x_hbm.shape[0]
    b_per_w = n_total // _NW
    n_chunks = b_per_w // _CHUNK
    wid = lax.axis_index("s") * _NC + lax.axis_index("c")
    base = wid * b_per_w

    for b in range(_NBUF):
        pltpu.async_copy(
            x_hbm.at[pl.ds(base + b * _CHUNK, _CHUNK)], idx_v.at[b], isems.at[b]
        )

    @pl.loop(0, n_chunks // _NBUF)
    def _outer(t):
        g0 = t * _NBUF
        for b in range(_NBUF):
            g = g0 + b
            off = base + g * _CHUNK

            @pl.when(t > 0)
            def _():
                pltpu.make_async_copy(
                    rows_v.at[b], out_hbm.at[pl.ds(base, _CHUNK)], osems.at[b]
                ).wait()

            pltpu.make_async_copy(
                x_hbm.at[pl.ds(off, _CHUNK)], idx_v.at[b], isems.at[b]
            ).wait()

            @pl.loop(0, _CHUNK // 16)
            def _remap(j):
                v = idx_v[b, pl.ds(j * 16, 16)]
                q = jnp.where(
                    v < _SPLIT,
                    v + v,
                    jnp.where(
                        v < _MID,
                        (v - _SPLIT) * 2 + 1,
                        _MID + (v - _MID) * 2,
                    ),
                )
                pidx_v[b, pl.ds(j * 16, 16)] = q

            pltpu.async_copy(
                table_hbm.at[pidx_v.at[b]], rows_v.at[b], gsems.at[b]
            ).wait()

            pltpu.async_copy(rows_v.at[b], out_hbm.at[pl.ds(off, _CHUNK)], osems.at[b])

            @pl.when(g + _NBUF < n_chunks)
            def _():
                off2 = base + (g + _NBUF) * _CHUNK
                pltpu.async_copy(
                    x_hbm.at[pl.ds(off2, _CHUNK)], idx_v.at[b], isems.at[b]
                )

    for b in range(_NBUF):
        pltpu.make_async_copy(
            rows_v.at[b], out_hbm.at[pl.ds(base, _CHUNK)], osems.at[b]
        ).wait()


@jax.jit
def _embed(flat_x, table_lin):
    n = flat_x.shape[0]
    k = pl.kernel(
        _embed_body,
        out_type=jax.ShapeDtypeStruct((n, _D), jnp.float32),
        mesh=plsc.VectorSubcoreMesh(core_axis_name="c", subcore_axis_name="s"),
        scratch_types=[
            pltpu.VMEM((_NBUF, _CHUNK), jnp.int32),
            pltpu.VMEM((_NBUF, _CHUNK), jnp.int32),
            pltpu.VMEM((_NBUF, _CHUNK, _D), jnp.float32),
            pltpu.SemaphoreType.DMA((_NBUF,)),
            pltpu.SemaphoreType.DMA((_NBUF,)),
            pltpu.SemaphoreType.DMA((_NBUF,)),
        ],
        compiler_params=pltpu.CompilerParams(use_tc_tiling_on_sc=False),
    )
    return k(flat_x, table_lin)


def kernel(x, embed_weight):
    b, h = x.shape
    table2 = _repack(embed_weight.T).reshape(2 * _PROWS, _D)
    out = _embed(x.reshape(b * h), table2)
    return out.reshape(b, h, _D)
